# Initial kernel scaffold; baseline (speedup 1.0000x reference)
import jax
import jax.numpy as jnp
from jax.experimental import pallas as pl


def kernel(x, lookup_table):
    raise NotImplementedError("write your pallas kernel here")


# speedup vs baseline: 1.1823x; 1.1823x over previous
import functools

import jax
import jax.numpy as jnp
from jax.experimental import pallas as pl
from jax.experimental.pallas import tpu as pltpu

K = 8192
DIM = 256
B, H, W = 4, 24, 24
M = B * H * W
KT = 512
NKT = K // KT


def _argmin_body(xf_ref, lt_ref, q_ref, xsq_ref, minv_ref, mini_ref):
    k = pl.program_id(0)

    @pl.when(k == 0)
    def _init():
        xf = xf_ref[...]
        xsq_ref[...] = jnp.sum(xf * xf, axis=1, keepdims=True)
        minv_ref[...] = jnp.full((M, 1), jnp.inf, jnp.float32)
        mini_ref[...] = jnp.zeros((M, 1), jnp.int32)

    lt = lt_ref[...]
    e = jax.lax.dot_general(
        xf_ref[...], lt,
        dimension_numbers=(((1,), (1,)), ((), ())),
        preferred_element_type=jnp.float32,
    )
    csq = jnp.sum(lt * lt, axis=1)
    d2 = (xsq_ref[...] - 2.0 * e) + csq[None, :]
    tmin = jnp.min(d2, axis=1, keepdims=True)
    gidx = k * KT + jax.lax.broadcasted_iota(jnp.int32, (M, KT), 1)
    tidx = jnp.min(
        jnp.where(d2 == tmin, gidx, jnp.int32(2**31 - 1)),
        axis=1, keepdims=True)
    upd = tmin < minv_ref[...]
    mini_ref[...] = jnp.where(upd, tidx, mini_ref[...])
    minv_ref[...] = jnp.where(upd, tmin, minv_ref[...])

    @pl.when(k == NKT - 1)
    def _out():
        q_ref[...] = mini_ref[...]


def _argmin_call(xf, lt):
    return pl.pallas_call(
        _argmin_body,
        grid=(NKT,),
        in_specs=[
            pl.BlockSpec((M, DIM), lambda k: (0, 0)),
            pl.BlockSpec((KT, DIM), lambda k: (k, 0)),
        ],
        out_specs=pl.BlockSpec((M, 1), lambda k: (0, 0)),
        out_shape=jax.ShapeDtypeStruct((M, 1), jnp.int32),
        scratch_shapes=[
            pltpu.VMEM((M, 1), jnp.float32),
            pltpu.VMEM((M, 1), jnp.float32),
            pltpu.VMEM((M, 1), jnp.int32),
        ],
        compiler_params=pltpu.CompilerParams(
            dimension_semantics=("arbitrary",),
        ),
    )(xf, lt)


def kernel(x, lookup_table):
    b, d, h, w = x.shape
    lt = lookup_table[0, 0]
    xf = jnp.transpose(x.reshape(b, d, h * w), (0, 2, 1)).reshape(M, DIM)
    q = _argmin_call(xf, lt)[:, 0]
    x_e_flat = jnp.take(lt, q, axis=0)
    x_e = jnp.transpose(x_e_flat.reshape(b, h * w, d), (0, 2, 1)).reshape(
        b, d, h, w)
    q_out = q.reshape(b, h, w)
    x_r = xf.reshape(b, d, h, w)
    codebook_loss = jnp.mean((x_r - x_e) ** 2)
    return x_e, q_out, codebook_loss


# speedup vs baseline: 1.4223x; 1.2030x over previous
import functools

import jax
import jax.numpy as jnp
from jax.experimental import pallas as pl
from jax.experimental.pallas import tpu as pltpu

K = 8192
DIM = 256
B, H, W = 4, 24, 24
M = B * H * W
KT = 2048
NKT = K // KT


def _argmin_body(xf_ref, lt_ref, q_ref, xsq_ref, minv_ref, mini_ref):
    k = pl.program_id(0)

    @pl.when(k == 0)
    def _init():
        xf = xf_ref[...]
        xsq_ref[...] = jnp.sum(xf * xf, axis=1, keepdims=True)

    ltm2 = lt_ref[...] * -2.0
    csq = jnp.sum(ltm2 * ltm2, axis=1) * 0.25
    e2 = jax.lax.dot_general(
        xf_ref[...], ltm2,
        dimension_numbers=(((1,), (1,)), ((), ())),
        preferred_element_type=jnp.float32,
    )
    d2 = (xsq_ref[...] + e2) + csq[None, :]
    tmin = jnp.min(d2, axis=1, keepdims=True)
    lidx = jax.lax.broadcasted_iota(jnp.int32, (M, KT), 1).astype(jnp.float32)
    tidx = jnp.min(
        jnp.where(d2 == tmin, lidx, jnp.inf),
        axis=1, keepdims=True) + jnp.float32(k * KT)
    upd = jnp.logical_or(tmin < minv_ref[...], k == 0)
    mini_ref[...] = jnp.where(upd, tidx, mini_ref[...])
    minv_ref[...] = jnp.where(upd, tmin, minv_ref[...])

    @pl.when(k == NKT - 1)
    def _out():
        q_ref[...] = mini_ref[...].astype(jnp.int32)


def _argmin_call(xf, lt):
    return pl.pallas_call(
        _argmin_body,
        grid=(NKT,),
        in_specs=[
            pl.BlockSpec((M, DIM), lambda k: (0, 0)),
            pl.BlockSpec((KT, DIM), lambda k: (k, 0)),
        ],
        out_specs=pl.BlockSpec((M, 1), lambda k: (0, 0)),
        out_shape=jax.ShapeDtypeStruct((M, 1), jnp.int32),
        scratch_shapes=[
            pltpu.VMEM((M, 1), jnp.float32),
            pltpu.VMEM((M, 1), jnp.float32),
            pltpu.VMEM((M, 1), jnp.float32),
        ],
        compiler_params=pltpu.CompilerParams(
            dimension_semantics=("arbitrary",),
        ),
    )(xf, lt)


def kernel(x, lookup_table):
    b, d, h, w = x.shape
    lt = lookup_table[0, 0]
    xf = jnp.transpose(x.reshape(b, d, h * w), (0, 2, 1)).reshape(M, DIM)
    q = _argmin_call(xf, lt)[:, 0]
    x_e_flat = jnp.take(lt, q, axis=0)
    x_e = jnp.transpose(x_e_flat.reshape(b, h * w, d), (0, 2, 1)).reshape(
        b, d, h, w)
    q_out = q.reshape(b, h, w)
    x_r = xf.reshape(b, d, h, w)
    codebook_loss = jnp.mean((x_r - x_e) ** 2)
    return x_e, q_out, codebook_loss


# speedup vs baseline: 1.4884x; 1.0465x over previous
import functools

import jax
import jax.numpy as jnp
from jax import lax
from jax.experimental import pallas as pl
from jax.experimental.pallas import tpu as pltpu
from jax.experimental.pallas import tpu_sc as plsc

K = 8192
DIM = 256
B, H, W = 4, 24, 24
M = B * H * W
KT = 2048
NKT = K // KT


def _argmin_body(xf_ref, lt_ref, q_ref, xsq_ref, minv_ref, mini_ref):
    k = pl.program_id(0)

    @pl.when(k == 0)
    def _init():
        xf = xf_ref[...]
        xsq_ref[...] = jnp.sum(xf * xf, axis=1, keepdims=True)

    ltm2 = lt_ref[...] * -2.0
    csq = jnp.sum(ltm2 * ltm2, axis=1) * 0.25
    e2 = jax.lax.dot_general(
        xf_ref[...], ltm2,
        dimension_numbers=(((1,), (1,)), ((), ())),
        preferred_element_type=jnp.float32,
    )
    d2 = (xsq_ref[...] + e2) + csq[None, :]
    tmin = jnp.min(d2, axis=1, keepdims=True)
    lidx = jax.lax.broadcasted_iota(jnp.int32, (M, KT), 1).astype(jnp.float32)
    tidx = jnp.min(
        jnp.where(d2 == tmin, lidx, jnp.inf),
        axis=1, keepdims=True) + jnp.float32(k * KT)
    upd = jnp.logical_or(tmin < minv_ref[...], k == 0)
    mini_ref[...] = jnp.where(upd, tidx, mini_ref[...])
    minv_ref[...] = jnp.where(upd, tmin, minv_ref[...])

    @pl.when(k == NKT - 1)
    def _out():
        q_ref[...] = mini_ref[...].astype(jnp.int32)


def _argmin_call(xf, lt):
    return pl.pallas_call(
        _argmin_body,
        grid=(NKT,),
        in_specs=[
            pl.BlockSpec((M, DIM), lambda k: (0, 0)),
            pl.BlockSpec((KT, DIM), lambda k: (k, 0)),
        ],
        out_specs=pl.BlockSpec((M, 1), lambda k: (0, 0)),
        out_shape=jax.ShapeDtypeStruct((M, 1), jnp.int32),
        scratch_shapes=[
            pltpu.VMEM((M, 1), jnp.float32),
            pltpu.VMEM((M, 1), jnp.float32),
            pltpu.VMEM((M, 1), jnp.float32),
        ],
        compiler_params=pltpu.CompilerParams(
            dimension_semantics=("arbitrary",),
        ),
    )(xf, lt)


NW = 32
BPW = M // NW


def _sc_gather_body(q_hbm, lt_hbm, out_hbm, idx_v, rows_v, sem):
    wid = lax.axis_index("s") * 2 + lax.axis_index("c")
    base = wid * BPW
    pltpu.sync_copy(q_hbm.at[pl.ds(base, BPW)], idx_v)
    pltpu.async_copy(lt_hbm.at[idx_v], rows_v, sem).wait()
    pltpu.sync_copy(rows_v, out_hbm.at[pl.ds(base, BPW)])


_sc_gather = functools.partial(
    pl.kernel,
    mesh=plsc.VectorSubcoreMesh(core_axis_name="c", subcore_axis_name="s"),
    out_type=jax.ShapeDtypeStruct((M, DIM), jnp.float32),
    scratch_types=[
        pltpu.VMEM((BPW,), jnp.int32),
        pltpu.VMEM((BPW, DIM), jnp.float32),
        pltpu.SemaphoreType.DMA,
    ],
)(_sc_gather_body)


def kernel(x, lookup_table):
    b, d, h, w = x.shape
    lt = lookup_table[0, 0]
    xf = jnp.transpose(x.reshape(b, d, h * w), (0, 2, 1)).reshape(M, DIM)
    q = _argmin_call(xf, lt)[:, 0]
    x_e_flat = _sc_gather(q, lt)
    x_e = jnp.transpose(x_e_flat.reshape(b, h * w, d), (0, 2, 1)).reshape(
        b, d, h, w)
    q_out = q.reshape(b, h, w)
    x_r = xf.reshape(b, d, h, w)
    codebook_loss = jnp.mean((x_r - x_e) ** 2)
    return x_e, q_out, codebook_loss
